# baseline (speedup 1.0000x reference)
import jax
import jax.numpy as jnp
from jax import lax
from jax.experimental import pallas as pl
from jax.experimental.pallas import tpu as pltpu
from jax.experimental.pallas import tpu_sc as plsc

N_TAB = 110000
N_REL = 64
DIM = 64
B = 65536
LAM = 1e-5

L = 16
NC = 2
NS = 16
NW = NC * NS
NB = B // NW
C = 128
NCHUNK = NB // C
TPC = C // L

_LN2 = 0.6931471805599453


def _rsqrt(s):
    s = jnp.maximum(s, 1e-24)
    bits = lax.bitcast_convert_type(s, jnp.int32)
    y = lax.bitcast_convert_type(jnp.int32(0x5F3759DF) - (bits >> 1), jnp.float32)
    for _ in range(3):
        y = y * (1.5 - 0.5 * s * y * y)
    return y


def _log(v):
    bits = lax.bitcast_convert_type(v, jnp.int32)
    e = ((bits >> 23) - 127).astype(jnp.float32)
    m = lax.bitcast_convert_type(
        (bits & jnp.int32(0x007FFFFF)) | jnp.int32(0x3F800000), jnp.float32)
    s = (m - 1.0) / (m + 1.0)
    s2 = s * s
    p = 1.0 / 9.0
    p = 1.0 / 7.0 + s2 * p
    p = 1.0 / 5.0 + s2 * p
    p = 1.0 / 3.0 + s2 * p
    p = 1.0 + s2 * p
    return e * _LN2 + 2.0 * s * p


def _softplus(x):
    u = jnp.exp(-jnp.abs(x))
    return jnp.maximum(x, 0.0) + _log(1.0 + u)


def _body(h_hbm, r_hbm, p_hbm, n_hbm, F, RF, out,
          idx_h, idx_r, idx_p, idx_n,
          rtF,
          bufs,
          st_v, sems):
    cid = lax.axis_index("c")
    sid = lax.axis_index("s")
    wid = sid * NC + cid
    base = wid * NB

    pltpu.sync_copy(RF, rtF)
    pltpu.sync_copy(h_hbm.at[pl.ds(base, NB)], idx_h)
    pltpu.sync_copy(r_hbm.at[pl.ds(base, NB)], idx_r)
    pltpu.sync_copy(p_hbm.at[pl.ds(base, NB)], idx_p)
    pltpu.sync_copy(n_hbm.at[pl.ds(base, NB)], idx_n)

    def fire(ci, slot):
        off = ci * C
        h_b, p_b, n_b = bufs[slot]
        sem = sems[slot]
        pltpu.async_copy(F.at[idx_h.at[pl.ds(off, C)]], h_b, sem)
        pltpu.async_copy(F.at[idx_p.at[pl.ds(off, C)]], p_b, sem)
        pltpu.async_copy(F.at[idx_n.at[pl.ds(off, C)]], n_b, sem)

    def wait_all(slot):
        h_b, p_b, n_b = bufs[slot]
        sem = sems[slot]
        for dst in (h_b, p_b, n_b):
            pltpu.make_async_copy(F.at[idx_h.at[pl.ds(0, C)]], dst, sem).wait()

    def fire_next(ci, slot):
        fire(jnp.minimum(ci, NCHUNK - 1), slot)

    iota = lax.iota(jnp.int32, L)

    def compute_chunk(ci, slot, kg, l2):
        h_b, p_b, n_b = bufs[slot]

        def tile(t, tc):
            kg2, l22 = tc
            row0 = t * L
            rows = iota + row0
            rvec = idx_r[pl.ds(ci * C + row0, L)]
            z = jnp.zeros((L,), jnp.float32)
            a_hh = a_pp = a_nn = z
            a_h2 = a_p2 = a_n2 = a_r2 = a_t2 = z
            a_ht = a_pt = a_nt = a_rt = z
            a_hr = a_pr = a_nr = a_hp = a_hn = z
            for d in range(DIM):
                col = (iota + d) & (DIM - 1)
                colT = col + DIM
                he = plsc.load_gather(h_b, [rows, col])
                hp = plsc.load_gather(h_b, [rows, colT])
                pe = plsc.load_gather(p_b, [rows, col])
                pp = plsc.load_gather(p_b, [rows, colT])
                ne = plsc.load_gather(n_b, [rows, col])
                nq = plsc.load_gather(n_b, [rows, colT])
                re = plsc.load_gather(rtF, [rvec, col])
                rp = plsc.load_gather(rtF, [rvec, colT])
                a_hh += he * hp
                a_pp += pe * pp
                a_nn += ne * nq
                a_h2 += he * he
                a_p2 += pe * pe
                a_n2 += ne * ne
                a_r2 += re * re
                a_t2 += rp * rp
                a_ht += he * rp
                a_pt += pe * rp
                a_nt += ne * rp
                a_rt += re * rp
                a_hr += he * re
                a_pr += pe * re
                a_nr += ne * re
                a_hp += he * pe
                a_hn += he * ne
            al, be, ga = a_hh, a_pp, a_nn
            s_a = a_h2 + 2.0 * al * a_ht + al * al * a_t2
            s_p = a_p2 + 2.0 * be * a_pt + be * be * a_t2
            s_n = a_n2 + 2.0 * ga * a_nt + ga * ga * a_t2
            s_r = a_r2
            d_ar = a_hr + al * a_rt
            d_ap = a_hp + be * a_ht + al * a_pt + al * be * a_t2
            d_an = a_hn + ga * a_ht + al * a_nt + al * ga * a_t2
            d_rp = a_pr + be * a_rt
            d_rn = a_nr + ga * a_rt
            ia = _rsqrt(s_a)
            ir = _rsqrt(s_r)
            ip = _rsqrt(s_p)
            iq = _rsqrt(s_n)
            ua = s_a * ia * ia
            ur = s_r * ir * ir
            up = s_p * ip * ip
            un = s_n * iq * iq
            c_ar = d_ar * ia * ir
            c_ap = d_ap * ia * ip
            c_an = d_an * ia * iq
            c_rp = d_rp * ir * ip
            c_rn = d_rn * ir * iq
            pos = ua + ur + up + 2.0 * (c_ar - c_ap - c_rp)
            neg = ua + ur + un + 2.0 * (c_ar - c_an - c_rn)
            sp = _softplus(pos - neg)
            return kg2 + sp, l22 + 0.5 * (ua + ur + up + un)

        return lax.fori_loop(0, TPC, tile, (kg, l2))

    z = jnp.zeros((L,), jnp.float32)

    fire(0, 0)

    def chunk_pair(cp, carry):
        kg, l2 = carry
        ci = cp * 2
        fire_next(ci + 1, 1)
        wait_all(0)
        kg, l2 = compute_chunk(ci, 0, kg, l2)
        fire_next(ci + 2, 0)
        wait_all(1)
        kg, l2 = compute_chunk(ci + 1, 1, kg, l2)
        return kg, l2

    kg, l2 = lax.fori_loop(0, NCHUNK // 2 - 1, chunk_pair, (z, z))
    fire(NCHUNK - 1, 1)
    wait_all(0)
    kg, l2 = compute_chunk(NCHUNK - 2, 0, kg, l2)
    wait_all(1)
    kg, l2 = compute_chunk(NCHUNK - 1, 1, kg, l2)

    st_v[pl.ds(0, L)] = kg
    st_v[pl.ds(L, L)] = l2
    pltpu.sync_copy(st_v, out.at[wid])


def kernel(h, r, pos_t, neg_t, entity_user_embed, ent_user_transfer,
           relation_embed, rel_transfer):
    mesh = plsc.VectorSubcoreMesh(core_axis_name="c", subcore_axis_name="s")

    def body(h_, r_, p_, n_, F, RF, out,
             idx_h, idx_r, idx_p, idx_n, rtF,
             b00, b01, b02, b10, b11, b12,
             st_v, sem0, sem1):
        bufs = ((b00, b01, b02), (b10, b11, b12))
        _body(h_, r_, p_, n_, F, RF, out,
              idx_h, idx_r, idx_p, idx_n, rtF,
              bufs, st_v, (sem0, sem1))

    f = pl.kernel(
        body,
        out_type=jax.ShapeDtypeStruct((NW, 8 * L), jnp.float32),
        mesh=mesh,
        compiler_params=pltpu.CompilerParams(
            needs_layout_passes=False, use_tc_tiling_on_sc=True),
        scratch_types=[
            pltpu.VMEM((NB,), jnp.int32),
            pltpu.VMEM((NB,), jnp.int32),
            pltpu.VMEM((NB,), jnp.int32),
            pltpu.VMEM((NB,), jnp.int32),
            pltpu.VMEM((N_REL, 2 * DIM), jnp.float32),
        ] + [pltpu.VMEM((C, 2 * DIM), jnp.float32)] * 6 + [
            pltpu.VMEM((8 * L,), jnp.float32),
            pltpu.SemaphoreType.DMA,
            pltpu.SemaphoreType.DMA,
        ],
    )
    fused = jnp.concatenate([entity_user_embed.T, ent_user_transfer.T],
                            axis=0).T
    rfused = jnp.concatenate([relation_embed.T, rel_transfer.T], axis=0).T
    part = f(h.astype(jnp.int32), r.astype(jnp.int32),
             pos_t.astype(jnp.int32), neg_t.astype(jnp.int32),
             fused, rfused)
    kg = jnp.sum(part[:, 0:L])
    l2 = jnp.sum(part[:, L:2 * L])
    return kg / B + LAM * (l2 / B)


# speedup vs baseline: 1.1525x; 1.1525x over previous
import jax
import jax.numpy as jnp
from jax import lax
from jax.experimental import pallas as pl
from jax.experimental.pallas import tpu as pltpu
from jax.experimental.pallas import tpu_sc as plsc

N_TAB = 110000
N_REL = 64
DIM = 64
B = 65536
LAM = 1e-5

L = 16
NC = 2
NS = 16
NW = NC * NS
NB = B // NW
C = 128
NCHUNK = NB // C
TPC = C // L

_LN2 = 0.6931471805599453


def _rsqrt(s):
    s = jnp.maximum(s, 1e-24)
    bits = lax.bitcast_convert_type(s, jnp.int32)
    y = lax.bitcast_convert_type(jnp.int32(0x5F3759DF) - (bits >> 1), jnp.float32)
    for _ in range(3):
        y = y * (1.5 - 0.5 * s * y * y)
    return y


def _log(v):
    bits = lax.bitcast_convert_type(v, jnp.int32)
    e = ((bits >> 23) - 127).astype(jnp.float32)
    m = lax.bitcast_convert_type(
        (bits & jnp.int32(0x007FFFFF)) | jnp.int32(0x3F800000), jnp.float32)
    s = (m - 1.0) / (m + 1.0)
    s2 = s * s
    p = 1.0 / 9.0
    p = 1.0 / 7.0 + s2 * p
    p = 1.0 / 5.0 + s2 * p
    p = 1.0 / 3.0 + s2 * p
    p = 1.0 + s2 * p
    return e * _LN2 + 2.0 * s * p


def _softplus(x):
    u = jnp.exp(-jnp.abs(x))
    return jnp.maximum(x, 0.0) + _log(1.0 + u)


def _body(h_hbm, r_hbm, p_hbm, n_hbm, F, RF, out,
          idx_h, idx_r, idx_p, idx_n,
          rtF,
          bufs,
          st_v, sems):
    cid = lax.axis_index("c")
    sid = lax.axis_index("s")
    wid = sid * NC + cid
    base = wid * NB

    pltpu.sync_copy(RF, rtF)
    pltpu.sync_copy(h_hbm.at[pl.ds(base, NB)], idx_h)
    pltpu.sync_copy(r_hbm.at[pl.ds(base, NB)], idx_r)
    pltpu.sync_copy(p_hbm.at[pl.ds(base, NB)], idx_p)
    pltpu.sync_copy(n_hbm.at[pl.ds(base, NB)], idx_n)

    def fire(ci, slot):
        off = ci * C
        h_b, p_b, n_b = bufs[slot]
        sem = sems[slot]
        pltpu.async_copy(F.at[idx_h.at[pl.ds(off, C)]], h_b, sem)
        pltpu.async_copy(F.at[idx_p.at[pl.ds(off, C)]], p_b, sem)
        pltpu.async_copy(F.at[idx_n.at[pl.ds(off, C)]], n_b, sem)

    def wait_all(slot):
        h_b, p_b, n_b = bufs[slot]
        sem = sems[slot]
        for dst in (h_b, p_b, n_b):
            pltpu.make_async_copy(F.at[idx_h.at[pl.ds(0, C)]], dst, sem).wait()

    def fire_next(ci, slot):
        fire(jnp.minimum(ci, NCHUNK - 1), slot)

    iota = lax.iota(jnp.int32, L)

    def compute_chunk(ci, slot, kg, l2):
        h_b, p_b, n_b = bufs[slot]

        def tile(t, tc):
            kg2, l22 = tc
            row0 = t * L
            rows = iota + row0
            rvec = idx_r[pl.ds(ci * C + row0, L)]
            z = jnp.zeros((L,), jnp.float32)
            a_hh = a_pp = a_nn = z
            a_h2 = a_p2 = a_n2 = a_r2 = a_t2 = z
            a_ht = a_pt = a_nt = a_rt = z
            a_hr = a_pr = a_nr = a_hp = a_hn = z
            for d in range(DIM):
                col = (iota + d) & (DIM - 1)
                colT = col + DIM
                he = plsc.load_gather(h_b, [rows, col])
                hp = plsc.load_gather(h_b, [rows, colT])
                pe = plsc.load_gather(p_b, [rows, col])
                pp = plsc.load_gather(p_b, [rows, colT])
                ne = plsc.load_gather(n_b, [rows, col])
                nq = plsc.load_gather(n_b, [rows, colT])
                re = plsc.load_gather(rtF, [rvec, col])
                rp = plsc.load_gather(rtF, [rvec, colT])
                a_hh += he * hp
                a_pp += pe * pp
                a_nn += ne * nq
                a_h2 += he * he
                a_p2 += pe * pe
                a_n2 += ne * ne
                a_r2 += re * re
                a_t2 += rp * rp
                a_ht += he * rp
                a_pt += pe * rp
                a_nt += ne * rp
                a_rt += re * rp
                a_hr += he * re
                a_pr += pe * re
                a_nr += ne * re
                a_hp += he * pe
                a_hn += he * ne
            al, be, ga = a_hh, a_pp, a_nn
            s_a = a_h2 + 2.0 * al * a_ht + al * al * a_t2
            s_p = a_p2 + 2.0 * be * a_pt + be * be * a_t2
            s_n = a_n2 + 2.0 * ga * a_nt + ga * ga * a_t2
            s_r = a_r2
            d_ar = a_hr + al * a_rt
            d_ap = a_hp + be * a_ht + al * a_pt + al * be * a_t2
            d_an = a_hn + ga * a_ht + al * a_nt + al * ga * a_t2
            d_rp = a_pr + be * a_rt
            d_rn = a_nr + ga * a_rt
            ia = _rsqrt(s_a)
            ir = _rsqrt(s_r)
            ip = _rsqrt(s_p)
            iq = _rsqrt(s_n)
            ua = s_a * ia * ia
            ur = s_r * ir * ir
            up = s_p * ip * ip
            un = s_n * iq * iq
            c_ar = d_ar * ia * ir
            c_ap = d_ap * ia * ip
            c_an = d_an * ia * iq
            c_rp = d_rp * ir * ip
            c_rn = d_rn * ir * iq
            pos = ua + ur + up + 2.0 * (c_ar - c_ap - c_rp)
            neg = ua + ur + un + 2.0 * (c_ar - c_an - c_rn)
            sp = _softplus(pos - neg)
            return kg2 + sp, l22 + 0.5 * (ua + ur + up + un)

        return lax.fori_loop(0, TPC, tile, (kg, l2))

    z = jnp.zeros((L,), jnp.float32)

    fire(0, 0)

    def chunk_pair(cp, carry):
        kg, l2 = carry
        ci = cp * 2
        fire_next(ci + 1, 1)
        wait_all(0)
        kg, l2 = compute_chunk(ci, 0, kg, l2)
        fire_next(ci + 2, 0)
        wait_all(1)
        kg, l2 = compute_chunk(ci + 1, 1, kg, l2)
        return kg, l2

    kg, l2 = lax.fori_loop(0, NCHUNK // 2 - 1, chunk_pair, (z, z))
    fire(NCHUNK - 1, 1)
    wait_all(0)
    kg, l2 = compute_chunk(NCHUNK - 2, 0, kg, l2)
    wait_all(1)
    kg, l2 = compute_chunk(NCHUNK - 1, 1, kg, l2)

    st_v[pl.ds(0, L)] = kg
    st_v[pl.ds(L, L)] = l2
    pltpu.sync_copy(st_v, out.at[wid])


def kernel(h, r, pos_t, neg_t, entity_user_embed, ent_user_transfer,
           relation_embed, rel_transfer):
    mesh = plsc.VectorSubcoreMesh(core_axis_name="c", subcore_axis_name="s")

    def body(h_, r_, p_, n_, F, RF, out,
             idx_h, idx_r, idx_p, idx_n, rtF,
             b00, b01, b02, b10, b11, b12,
             st_v, sem0, sem1):
        bufs = ((b00, b01, b02), (b10, b11, b12))
        _body(h_, r_, p_, n_, F, RF, out,
              idx_h, idx_r, idx_p, idx_n, rtF,
              bufs, st_v, (sem0, sem1))

    f = pl.kernel(
        body,
        out_type=jax.ShapeDtypeStruct((NW, 8 * L), jnp.float32),
        mesh=mesh,
        compiler_params=pltpu.CompilerParams(
            needs_layout_passes=False, use_tc_tiling_on_sc=True),
        scratch_types=[
            pltpu.VMEM((NB,), jnp.int32),
            pltpu.VMEM((NB,), jnp.int32),
            pltpu.VMEM((NB,), jnp.int32),
            pltpu.VMEM((NB,), jnp.int32),
            pltpu.VMEM((N_REL, 2 * DIM), jnp.float32),
        ] + [pltpu.VMEM((C, 2 * DIM), jnp.float32)] * 6 + [
            pltpu.VMEM((8 * L,), jnp.float32),
            pltpu.SemaphoreType.DMA,
            pltpu.SemaphoreType.DMA,
        ],
    )
    RB = 2048

    def _fuse_body(et_ref, tt_ref, o_ref):
        o_ref[:, 0:DIM] = et_ref[...].T
        o_ref[:, DIM:2 * DIM] = tt_ref[...].T

    fused = pl.pallas_call(
        _fuse_body,
        out_shape=jax.ShapeDtypeStruct((N_TAB, 2 * DIM), jnp.float32),
        grid=(pl.cdiv(N_TAB, RB),),
        in_specs=[pl.BlockSpec((DIM, RB), lambda i: (0, i)),
                  pl.BlockSpec((DIM, RB), lambda i: (0, i))],
        out_specs=pl.BlockSpec((RB, 2 * DIM), lambda i: (i, 0)),
    )(entity_user_embed.T, ent_user_transfer.T)
    rfused = jnp.concatenate([relation_embed.T, rel_transfer.T], axis=0).T
    part = f(h.astype(jnp.int32), r.astype(jnp.int32),
             pos_t.astype(jnp.int32), neg_t.astype(jnp.int32),
             fused, rfused)
    kg = jnp.sum(part[:, 0:L])
    l2 = jnp.sum(part[:, L:2 * L])
    return kg / B + LAM * (l2 / B)
